# baseline (speedup 1.0000x reference)
import jax
import jax.numpy as jnp
from jax.experimental import pallas as pl


def _qjl_body(d_ref, m_ref, p_ref, plo_ref, pin_ref, pout_ref, oin_ref, oout_ref):
    bb, n, d = d_ref.shape
    x = d_ref[...]
    m = m_ref[...]
    xo = (x * m).reshape(bb * n, d)
    xi = x.reshape(bb * n, d) - xo
    dn = (((1,), (1,)), ((), ()))
    si = jax.lax.dot_general(xi, p_ref[...], dn, preferred_element_type=jnp.float32)
    so = jax.lax.dot_general(xo, plo_ref[...], dn, preferred_element_type=jnp.float32)
    bi = (si > 0).astype(jnp.int8)
    bo = (so > 0).astype(jnp.int8)
    pi = jnp.dot(bi, pin_ref[...], preferred_element_type=jnp.int32)
    po = jnp.dot(bo, pout_ref[...], preferred_element_type=jnp.int32)
    oin_ref[...] = pi.astype(jnp.uint8).reshape(bb, n, pi.shape[-1])
    oout_ref[...] = po.astype(jnp.uint8).reshape(bb, n, po.shape[-1])


def kernel(data, outlier_mask, proj_dir_quant):
    b, h, blk, n, d = data.shape
    s = proj_dir_quant.shape[0]
    s_lo = (s // 16) * 8
    g = b * h * blk

    data3 = data.reshape(g, n, d)
    mask3 = outlier_mask.astype(jnp.float32).reshape(g, 1, d)
    proj = proj_dir_quant
    proj_lo = proj[:s_lo]

    rows = jnp.arange(s)
    pack_in = jnp.where(
        (rows[:, None] // 8) == jnp.arange(s // 8)[None, :],
        jnp.left_shift(1, rows[:, None] % 8),
        0,
    ).astype(jnp.int8)
    pack_out = pack_in[:s_lo, : s_lo // 8]

    bsz = 64
    oi, oo = pl.pallas_call(
        _qjl_body,
        grid=(g // bsz,),
        in_specs=[
            pl.BlockSpec((bsz, n, d), lambda i: (i, 0, 0)),
            pl.BlockSpec((bsz, 1, d), lambda i: (i, 0, 0)),
            pl.BlockSpec((s, d), lambda i: (0, 0)),
            pl.BlockSpec((s_lo, d), lambda i: (0, 0)),
            pl.BlockSpec((s, s // 8), lambda i: (0, 0)),
            pl.BlockSpec((s_lo, s_lo // 8), lambda i: (0, 0)),
        ],
        out_specs=[
            pl.BlockSpec((bsz, n, s // 8), lambda i: (i, 0, 0)),
            pl.BlockSpec((bsz, n, s_lo // 8), lambda i: (i, 0, 0)),
        ],
        out_shape=[
            jax.ShapeDtypeStruct((g, n, s // 8), jnp.uint8),
            jax.ShapeDtypeStruct((g, n, s_lo // 8), jnp.uint8),
        ],
    )(data3, mask3, proj, proj_lo, pack_in, pack_out)

    hash_in = oi.reshape(b, h, blk, n, s // 8)
    hash_out = oo.reshape(b, h, blk, n, s_lo // 8)
    return (hash_in, hash_out)


# speedup vs baseline: 1.0026x; 1.0026x over previous
import jax
import jax.numpy as jnp
from jax.experimental import pallas as pl


def _qjl_body(d_ref, m_ref, p_ref, plo_ref, pin_ref, pout_ref, oin_ref, oout_ref):
    bb, n, d = d_ref.shape
    x = d_ref[...]
    m = m_ref[...]
    xo = (x * m).reshape(bb * n, d)
    xi = x.reshape(bb * n, d) - xo
    dn = (((1,), (1,)), ((), ()))
    si = jax.lax.dot_general(xi, p_ref[...], dn, preferred_element_type=jnp.float32)
    so = jax.lax.dot_general(xo, plo_ref[...], dn, preferred_element_type=jnp.float32)
    bi = (si > 0).astype(jnp.int8)
    bo = (so > 0).astype(jnp.int8)
    pi = jnp.dot(bi, pin_ref[...], preferred_element_type=jnp.int32)
    po = jnp.dot(bo, pout_ref[...], preferred_element_type=jnp.int32)
    oin_ref[...] = pi.astype(jnp.uint8).reshape(bb, n, pi.shape[-1])
    oout_ref[...] = po.astype(jnp.uint8).reshape(bb, n, po.shape[-1])


def kernel(data, outlier_mask, proj_dir_quant):
    b, h, blk, n, d = data.shape
    s = proj_dir_quant.shape[0]
    s_lo = (s // 16) * 8
    g = b * h * blk

    data3 = data.reshape(g, n, d)
    mask3 = outlier_mask.astype(jnp.float32).reshape(g, 1, d)
    proj = proj_dir_quant
    proj_lo = proj[:s_lo]

    rows = jnp.arange(s)
    pack_in = jnp.where(
        (rows[:, None] // 8) == jnp.arange(s // 8)[None, :],
        jnp.left_shift(1, rows[:, None] % 8),
        0,
    ).astype(jnp.int8)
    pack_out = pack_in[:s_lo, : s_lo // 8]

    bsz = 128
    oi, oo = pl.pallas_call(
        _qjl_body,
        grid=(g // bsz,),
        in_specs=[
            pl.BlockSpec((bsz, n, d), lambda i: (i, 0, 0)),
            pl.BlockSpec((bsz, 1, d), lambda i: (i, 0, 0)),
            pl.BlockSpec((s, d), lambda i: (0, 0)),
            pl.BlockSpec((s_lo, d), lambda i: (0, 0)),
            pl.BlockSpec((s, s // 8), lambda i: (0, 0)),
            pl.BlockSpec((s_lo, s_lo // 8), lambda i: (0, 0)),
        ],
        out_specs=[
            pl.BlockSpec((bsz, n, s // 8), lambda i: (i, 0, 0)),
            pl.BlockSpec((bsz, n, s_lo // 8), lambda i: (i, 0, 0)),
        ],
        out_shape=[
            jax.ShapeDtypeStruct((g, n, s // 8), jnp.uint8),
            jax.ShapeDtypeStruct((g, n, s_lo // 8), jnp.uint8),
        ],
    )(data3, mask3, proj, proj_lo, pack_in, pack_out)

    hash_in = oi.reshape(b, h, blk, n, s // 8)
    hash_out = oo.reshape(b, h, blk, n, s_lo // 8)
    return (hash_in, hash_out)
